# baseline (speedup 1.0000x reference)
import functools

import jax
import jax.numpy as jnp
from jax import lax
from jax.experimental import pallas as pl
from jax.experimental.pallas import tpu as pltpu
from jax.experimental.pallas import tpu_sc as plsc

_N = 65536
_NW = 32
_PER_W = _N // _NW


def _sc_body(x_hbm, o_hbm, sem0, sem1):
    wid = lax.axis_index("s") * 2 + lax.axis_index("c")
    base = wid * _PER_W
    cp0 = pltpu.async_copy(
        x_hbm.at[pl.ds(base, _PER_W), 1], o_hbm.at[pl.ds(base, _PER_W), 0], sem0
    )
    cp1 = pltpu.async_copy(
        x_hbm.at[pl.ds(base, _PER_W), 0], o_hbm.at[pl.ds(base, _PER_W), 1], sem1
    )
    cp0.wait()
    cp1.wait()


_sc_call = functools.partial(
    pl.kernel,
    mesh=plsc.VectorSubcoreMesh(core_axis_name="c", subcore_axis_name="s"),
    out_type=jax.ShapeDtypeStruct((_N, 2, 2, 128), jnp.float32),
    scratch_types=[
        pltpu.SemaphoreType.DMA,
        pltpu.SemaphoreType.DMA,
    ],
)(_sc_body)


def kernel(x):
    a, b, c, s, t = x.shape
    g, l = c // 128, 128
    xr = (
        x.transpose(0, 1, 3, 4, 2)
        .reshape(a, b, s, t, g, l)
        .transpose(0, 1, 2, 4, 3, 5)
        .reshape(_N, 2, 2, l)
    )
    out = _sc_call(xr)
    return (
        out.reshape(a, b, s, g, t, l)
        .transpose(0, 1, 2, 4, 3, 5)
        .reshape(a, b, s, t, c)
        .transpose(0, 1, 4, 2, 3)
    )
